# baseline (speedup 1.0000x reference)
import functools
import math

import jax
import jax.numpy as jnp
from jax import lax
from jax.experimental import pallas as pl
from jax.experimental.pallas import tpu as pltpu
from jax.experimental.pallas import tpu_sc as plsc

V = 32000
D = 768
H = 12
DH = 64
L = 2
E = 8
TOPK = 2
T = 2048
CAP = 512
NSLOT = E * CAP
ESTRIDE = 576
ZROW = 512
NPAD = E * ESTRIDE
BQ = 256
SCALE = 1.0 / math.sqrt(DH)

NC, NS = 2, 16
NW = NC * NS


def _ln(x, g, b):
    m = jnp.mean(x, axis=-1, keepdims=True)
    v = jnp.mean((x - m) ** 2, axis=-1, keepdims=True)
    return (x - m) / jnp.sqrt(v + 1e-5) * g + b


def _qkv_body(x_ref, g_ref, b_ref, w_ref, o_ref):
    h = _ln(x_ref[...], g_ref[...], b_ref[...])
    o_ref[...] = jnp.dot(h, w_ref[...], preferred_element_type=jnp.float32)


def _qkv_call(x, g, b, w):
    return pl.pallas_call(
        _qkv_body,
        grid=(T // BQ,),
        in_specs=[
            pl.BlockSpec((BQ, D), lambda i: (i, 0)),
            pl.BlockSpec((1, D), lambda i: (0, 0)),
            pl.BlockSpec((1, D), lambda i: (0, 0)),
            pl.BlockSpec((D, 3 * D), lambda i: (0, 0)),
        ],
        out_specs=pl.BlockSpec((BQ, 3 * D), lambda i: (i, 0)),
        out_shape=jax.ShapeDtypeStruct((T, 3 * D), jnp.float32),
    )(x, g, b, w)


def _attn_body(q_ref, k_ref, v_ref, sin_ref, cos_ref, sq_ref, cq_ref, o_ref):
    qb = pl.program_id(1)
    q = q_ref[0]
    k = k_ref[0]
    v = v_ref[0]
    sin = sin_ref[...]
    cos = cos_ref[...]
    sq = sq_ref[...]
    cq = cq_ref[...]
    q1, q2 = q[:, : DH // 2], q[:, DH // 2 :]
    qr = jnp.concatenate([q1 * cq - q2 * sq, q2 * cq + q1 * sq], axis=1)
    k1, k2 = k[:, : DH // 2], k[:, DH // 2 :]
    kr = jnp.concatenate([k1 * cos - k2 * sin, k2 * cos + k1 * sin], axis=1)
    s = lax.dot_general(qr, kr, (((1,), (1,)), ((), ())),
                        preferred_element_type=jnp.float32) * SCALE
    rows = qb * BQ + lax.broadcasted_iota(jnp.int32, (BQ, T), 0)
    cols = lax.broadcasted_iota(jnp.int32, (BQ, T), 1)
    s = jnp.where(cols <= rows, s, -1e30)
    m = jnp.max(s, axis=1, keepdims=True)
    p = jnp.exp(s - m)
    p = p / jnp.sum(p, axis=1, keepdims=True)
    o_ref[0] = jnp.dot(p, v, preferred_element_type=jnp.float32)


def _attn_call(q, k, v, sin, cos):
    return pl.pallas_call(
        _attn_body,
        grid=(H, T // BQ),
        in_specs=[
            pl.BlockSpec((1, BQ, DH), lambda h, i: (h, i, 0)),
            pl.BlockSpec((1, T, DH), lambda h, i: (h, 0, 0)),
            pl.BlockSpec((1, T, DH), lambda h, i: (h, 0, 0)),
            pl.BlockSpec((T, DH // 2), lambda h, i: (0, 0)),
            pl.BlockSpec((T, DH // 2), lambda h, i: (0, 0)),
            pl.BlockSpec((BQ, DH // 2), lambda h, i: (i, 0)),
            pl.BlockSpec((BQ, DH // 2), lambda h, i: (i, 0)),
        ],
        out_specs=pl.BlockSpec((1, BQ, DH), lambda h, i: (h, i, 0)),
        out_shape=jax.ShapeDtypeStruct((H, T, DH), jnp.float32),
    )(q, k, v, sin, cos, sin, cos)


def _proj_body(x_ref, w_ref, o_ref):
    o_ref[...] = jnp.dot(x_ref[...], w_ref[...],
                         preferred_element_type=jnp.float32)


def _proj_call(ao, w):
    return pl.pallas_call(
        _proj_body,
        grid=(T // BQ,),
        in_specs=[pl.BlockSpec((BQ, D), lambda i: (i, 0)),
                  pl.BlockSpec((D, D), lambda i: (0, 0))],
        out_specs=pl.BlockSpec((BQ, D), lambda i: (i, 0)),
        out_shape=jax.ShapeDtypeStruct((T, D), jnp.float32),
    )(ao, w)


def _route_body(x_ref, a_ref, g2_ref, b2_ref, wr_ref, br_ref,
                wn_ref, bn_ref, eps_ref,
                x2_ref, h2_ref, sel_ref, src_ref, wsl_ref):
    x2 = x_ref[...] + a_ref[...]
    x2_ref[...] = x2
    h2 = _ln(x2, g2_ref[...], b2_ref[...])
    h2_ref[...] = h2
    lg = jnp.dot(h2, wr_ref[...], preferred_element_type=jnp.float32) + br_ref[...]
    pre = jnp.dot(h2, wn_ref[...], preferred_element_type=jnp.float32) + bn_ref[...]
    noisy = lg + eps_ref[...] * jax.nn.softplus(pre)

    e_iota = lax.broadcasted_iota(jnp.int32, (T, E), 1)
    m0 = jnp.max(noisy, axis=1, keepdims=True)
    ix0 = jnp.min(jnp.where(noisy == m0, e_iota, E), axis=1, keepdims=True)
    n1 = jnp.where(e_iota == ix0, -jnp.inf, noisy)
    m1 = jnp.max(n1, axis=1, keepdims=True)
    ix1 = jnp.min(jnp.where(n1 == m1, e_iota, E), axis=1, keepdims=True)
    ez = jnp.exp(m1 - m0)
    z = 1.0 + ez
    w0 = 1.0 / z
    w1 = ez / z

    mask = jnp.logical_or(e_iota == ix0, e_iota == ix1).astype(jnp.int32)
    c = mask
    sh = 1
    while sh < T:
        c = c + jnp.concatenate(
            [jnp.zeros((sh, E), jnp.int32), c[: T - sh]], axis=0)
        sh *= 2

    cnt0 = jnp.sum(jnp.where(e_iota == ix0, c, 0), axis=1, keepdims=True) - 1
    cnt1 = jnp.sum(jnp.where(e_iota == ix1, c, 0), axis=1, keepdims=True) - 1
    v0 = cnt0 < CAP
    v1 = cnt1 < CAP
    src0 = jnp.where(v0, ix0 * ESTRIDE + cnt0, ZROW)
    src1 = jnp.where(v1, ix1 * ESTRIDE + cnt1, ZROW)
    lo_first = ix0 < ix1
    src_lo = jnp.where(lo_first, src0, src1)
    src_hi = jnp.where(lo_first, src1, src0)
    src_ref[...] = jnp.concatenate([src_lo, src_hi], axis=1)

    pdense = (jnp.where(e_iota == ix0, w0, 0.0)
              + jnp.where(e_iota == ix1, w1, 0.0))
    t_iota = lax.broadcasted_iota(jnp.int32, (T, CAP), 0)
    j_iota = lax.broadcasted_iota(jnp.int32, (T, CAP), 1)
    for e in range(E):
        hit = jnp.logical_and(c[:, e : e + 1] - 1 == j_iota,
                              mask[:, e : e + 1] > 0)
        sel_ref[e, :] = jnp.sum(jnp.where(hit, t_iota, 0), axis=0)
        wsl_ref[e, :] = jnp.sum(jnp.where(hit, pdense[:, e : e + 1], 0.0),
                                axis=0)


def _route_call(x, a, g2, b2, wr, br, wn, bn, eps):
    def full(shp):
        return pl.BlockSpec(shp, lambda: tuple(0 for _ in shp))

    return pl.pallas_call(
        _route_body,
        in_specs=[
            full((T, D)), full((T, D)),
            full((1, D)), full((1, D)),
            full((D, E)), full((1, E)), full((D, E)), full((1, E)),
            full((T, E)),
        ],
        out_specs=[
            full((T, D)), full((T, D)), full((E, CAP)),
            full((T, 2)), full((E, CAP)),
        ],
        out_shape=[
            jax.ShapeDtypeStruct((T, D), jnp.float32),
            jax.ShapeDtypeStruct((T, D), jnp.float32),
            jax.ShapeDtypeStruct((E, CAP), jnp.int32),
            jax.ShapeDtypeStruct((T, 2), jnp.int32),
            jax.ShapeDtypeStruct((E, CAP), jnp.float32),
        ],
    )(x, a, g2, b2, wr, br, wn, bn, eps)


def _mlp_body(x_ref, w1_ref, b1_ref, w2_ref, b2_ref, ws_ref, o_ref):
    h = jnp.maximum(
        jnp.dot(x_ref[...], w1_ref[0], preferred_element_type=jnp.float32)
        + b1_ref[0], 0.0)
    o = (jnp.dot(h, w2_ref[0], preferred_element_type=jnp.float32)
         + b2_ref[0])
    o_ref[:CAP, :] = o * ws_ref[0, 0][:, None]
    o_ref[CAP:, :] = jnp.zeros((ESTRIDE - CAP, D), jnp.float32)


def _mlp_call(xe, w1, b1, w2, b2, wsl):
    return pl.pallas_call(
        _mlp_body,
        grid=(E,),
        in_specs=[
            pl.BlockSpec((CAP, D), lambda e: (e, 0)),
            pl.BlockSpec((1, D, 4 * D), lambda e: (e, 0, 0)),
            pl.BlockSpec((1, 1, 4 * D), lambda e: (e, 0, 0)),
            pl.BlockSpec((1, 4 * D, D), lambda e: (e, 0, 0)),
            pl.BlockSpec((1, 1, D), lambda e: (e, 0, 0)),
            pl.BlockSpec((1, 1, CAP), lambda e: (e, 0, 0)),
        ],
        out_specs=pl.BlockSpec((ESTRIDE, D), lambda e: (e, 0)),
        out_shape=jax.ShapeDtypeStruct((NPAD, D), jnp.float32),
    )(xe, w1, b1, w2, b2, wsl.reshape(E, 1, CAP))


def _add_body(a_ref, b_ref, o_ref):
    o_ref[...] = a_ref[...] + b_ref[...]


def _add_call(a, b):
    return pl.pallas_call(
        _add_body,
        grid=(T // BQ,),
        in_specs=[pl.BlockSpec((BQ, D), lambda i: (i, 0)),
                  pl.BlockSpec((BQ, D), lambda i: (i, 0))],
        out_specs=pl.BlockSpec((BQ, D), lambda i: (i, 0)),
        out_shape=jax.ShapeDtypeStruct((T, D), jnp.float32),
    )(a, b)


def _lnf_body(x_ref, g_ref, b_ref, o_ref):
    o_ref[...] = _ln(x_ref[...], g_ref[...], b_ref[...])


def _lnf_call(x, g, b):
    return pl.pallas_call(
        _lnf_body,
        grid=(T // BQ,),
        in_specs=[pl.BlockSpec((BQ, D), lambda i: (i, 0)),
                  pl.BlockSpec((1, D), lambda i: (0, 0)),
                  pl.BlockSpec((1, D), lambda i: (0, 0))],
        out_specs=pl.BlockSpec((BQ, D), lambda i: (i, 0)),
        out_shape=jax.ShapeDtypeStruct((T, D), jnp.float32),
    )(x, g, b)


@functools.cache
def _sc_mesh():
    return plsc.VectorSubcoreMesh(core_axis_name="c", subcore_axis_name="s",
                                  num_cores=NC, num_subcores=NS)


def _wid():
    return lax.axis_index("s") * NC + lax.axis_index("c")


@functools.cache
def _gather_kernel(nrows, per_w):
    @functools.partial(
        pl.kernel, mesh=_sc_mesh(),
        out_type=jax.ShapeDtypeStruct((nrows, D), jnp.float32),
        scratch_types=[pltpu.VMEM((per_w,), jnp.int32),
                       pltpu.VMEM((per_w, D), jnp.float32),
                       pltpu.SemaphoreType.DMA],
    )
    def _g(table_hbm, idx_hbm, out_hbm, idx_v, rows_v, sem):
        base = _wid() * per_w
        pltpu.sync_copy(idx_hbm.at[pl.ds(base, per_w)], idx_v)
        pltpu.async_copy(table_hbm.at[idx_v], rows_v, sem).wait()
        pltpu.sync_copy(rows_v, out_hbm.at[pl.ds(base, per_w)])

    return _g


def _emb_gather(table, ids):
    return _gather_kernel(T, T // NW)(table, ids)


def _dispatch_gather(h2, sel):
    return _gather_kernel(NSLOT, NSLOT // NW)(h2, sel)


@functools.cache
def _combine_kernel():
    @functools.partial(
        pl.kernel, mesh=_sc_mesh(),
        out_type=jax.ShapeDtypeStruct((T, D), jnp.float32),
        scratch_types=[pltpu.VMEM((64,), jnp.int32),
                       pltpu.VMEM((64, D), jnp.float32),
                       pltpu.VMEM((32, D), jnp.float32),
                       pltpu.SemaphoreType.DMA],
    )
    def _c(x2_hbm, o_hbm, src_hbm, out_hbm, idx_v, rows_v, acc_v, sem):
        for half in range(2):
            tok = _wid() * 64 + half * 32
            pltpu.sync_copy(src_hbm.at[pl.ds(2 * tok, 64)], idx_v)
            pltpu.sync_copy(x2_hbm.at[pl.ds(tok, 32)], acc_v)
            pltpu.async_copy(o_hbm.at[idx_v], rows_v, sem).wait()

            def jbody(j, _):
                def ibody(i, _):
                    sl = pl.ds(i * 16, 16)
                    y = rows_v[2 * j, sl] + rows_v[2 * j + 1, sl]
                    acc_v[j, sl] = acc_v[j, sl] + y
                    return 0

                lax.fori_loop(0, D // 16, ibody, 0)
                return 0

            lax.fori_loop(0, 32, jbody, 0)
            pltpu.sync_copy(acc_v, out_hbm.at[pl.ds(tok, 32)])

    return _c


def _combine(x2, o, src):
    return _combine_kernel()(x2, o, src)


def _sin_cos():
    pos = jnp.arange(T, dtype=jnp.float32)[:, None]
    inv = jnp.exp(jnp.arange(0, DH, 2, dtype=jnp.float32)
                  * (-math.log(10000.0) / DH))
    return jnp.sin(pos * inv), jnp.cos(pos * inv)


def kernel(params, input_ids):
    p = params
    ids = input_ids.reshape(T).astype(jnp.int32)
    tok = _emb_gather(p['tok_emb'], ids)
    x = _add_call(tok, p['pos_emb'])
    sin, cos = _sin_cos()
    nkey = jax.random.key(42)
    for l in range(L):
        qkv = _qkv_call(x, p['ln1_g'][l][None, :], p['ln1_b'][l][None, :],
                        p['Wqkv'][l])
        qkv4 = qkv.reshape(T, 3, H, DH).transpose(1, 2, 0, 3)
        ao = _attn_call(qkv4[0], qkv4[1], qkv4[2], sin, cos).reshape(T, D)
        a = _proj_call(ao, p['Wproj'][l])
        eps = jax.random.normal(jax.random.fold_in(nkey, l), (1, T, E),
                                dtype=jnp.float32).reshape(T, E)
        x2, h2, sel, src, wsl = _route_call(
            x, a, p['ln2_g'][l][None, :], p['ln2_b'][l][None, :],
            p['Wr'][l], p['br'][l][None, :],
            p['Wn'][l], p['bn'][l][None, :], eps)
        xe = _dispatch_gather(h2, sel.reshape(NSLOT))
        o = _mlp_call(xe, p['We1'][l], p['be1'][l][:, None, :],
                      p['We2'][l], p['be2'][l][:, None, :], wsl)
        x = _combine(x2, o, src.reshape(2 * T))
    out = _lnf_call(x, p['lnf_g'][None, :], p['lnf_b'][None, :])
    return out.reshape(1, T, D)
